# baseline (speedup 1.0000x reference)
import functools

import jax
import jax.numpy as jnp
from jax import lax
from jax.experimental import pallas as pl
from jax.experimental.pallas import tpu as pltpu
from jax.experimental.pallas import tpu_sc as plsc

B = 32
NA = 3
C85 = 85
G = 76
S = G * G
CH = NA * C85
STRIDE = 8.0
AW = (10.0, 16.0, 33.0)
AH = (13.0, 30.0, 23.0)

K = 304
NCHUNK = S // K
SUB = K // 16

def _cb_consts(ch_vec):
    c_vec = ch_vec % C85
    a_vec = ch_vec // C85
    m0 = c_vec == 0
    m01 = c_vec <= 1
    m23 = (c_vec >= 2) & (c_vec <= 3)
    aw = jnp.where(a_vec == 0, AW[0], jnp.where(a_vec == 1, AW[1], AW[2]))
    ah = jnp.where(a_vec == 0, AH[0], jnp.where(a_vec == 1, AH[1], AH[2]))
    anch = jnp.where(m23, jnp.where(c_vec == 2, aw, ah), 1.0)
    return m0, m01, m23, anch


def _sc_body(x_hbm, out_hbm, in_vmem, stage):
    cid = lax.axis_index("c")
    sid = lax.axis_index("s")
    b = sid * 2 + cid

    def chunk_body(ci, carry):
        s0 = ci * K
        pltpu.sync_copy(x_hbm.at[b, :, pl.ds(s0, K)], in_vmem)

        def t_body(t, carry2):
            def srel_body(s_rel, carry3):
                sloc = t * 16 + s_rel
                s_idx = s0 + sloc
                gxf = ((s_idx % G) * 1.0).astype(jnp.float32)
                gyf = ((s_idx // G) * 1.0).astype(jnp.float32)
                s_vec = jnp.full((16,), sloc, jnp.int32)
                for cb in range(16):
                    ch_vec = cb * 16 + lax.iota(jnp.int32, 16)
                    if cb == 15:
                        ch_vec = jnp.minimum(ch_vec, CH - 1)
                    v = plsc.load_gather(in_vmem, [ch_vec, s_vec])
                    sig = 1.0 / (1.0 + jnp.exp(-v))
                    special = cb in (0, 5, 10)
                    if special:
                        m0, m01, m23, anch = _cb_consts(cb * 16 + lax.iota(jnp.int32, 16))
                        ex = jnp.exp(v)
                        gvec = jnp.where(m0, gxf, gyf)
                        box01 = (sig + gvec) * STRIDE
                        box23 = ex * anch
                        res = jnp.where(m01, box01, jnp.where(m23, box23, sig))
                    else:
                        res = sig
                    if cb == 15:
                        row_vec = jnp.full((16,), s_rel, jnp.int32)
                        lane = lax.iota(jnp.int32, 16)
                        plsc.store_scatter(
                            stage, [row_vec, jnp.minimum(cb * 16 + lane, CH - 1)],
                            res, mask=lane < 15,
                        )
                    else:
                        stage[s_rel, pl.ds(cb * 16, 16)] = res
                return carry3

            lax.fori_loop(0, 16, srel_body, 0)
            pltpu.sync_copy(stage, out_hbm.at[b, ci * SUB + t])
            return carry2

        lax.fori_loop(0, SUB, t_body, 0)
        return carry

    lax.fori_loop(0, NCHUNK, chunk_body, 0)


_sc_run = functools.partial(
    pl.kernel,
    mesh=plsc.VectorSubcoreMesh(core_axis_name="c", subcore_axis_name="s"),
    out_type=jax.ShapeDtypeStruct((B, S // 16, 16, CH), jnp.float32),
    scratch_types=[
        pltpu.VMEM((CH, K), jnp.float32),
        pltpu.VMEM((16, CH), jnp.float32),
    ],
    compiler_params=pltpu.CompilerParams(
        use_tc_tiling_on_sc=False, needs_layout_passes=False
    ),
)(_sc_body)


def kernel(x):
    out = _sc_run(x.reshape(B, CH, S))
    return out.reshape(B, S * NA, C85), 0.0


# speedup vs baseline: 1.6410x; 1.6410x over previous
import functools

import jax
import jax.numpy as jnp
from jax import lax
from jax.experimental import pallas as pl
from jax.experimental.pallas import tpu as pltpu
from jax.experimental.pallas import tpu_sc as plsc

B = 32
NA = 3
C85 = 85
G = 76
S = G * G
CH = NA * C85
STRIDE = 8.0
AW = (10.0, 16.0, 33.0)
AH = (13.0, 30.0, 23.0)

K = 304
NCHUNK = S // K
SUB = K // 16
TILE_W = 4080


def _sigmoid(v):
    return 1.0 / (1.0 + jnp.exp(-v))


def _sc_body(x_hbm, out_hbm, in_vmem, stage2, sems):
    cid = lax.axis_index("c")
    sid = lax.axis_index("s")
    b = sid * 2 + cid

    lane = lax.iota(jnp.int32, 16)
    base_idx = lane * CH

    def chunk_body(ci, carry):
        s0 = ci * K
        pltpu.sync_copy(x_hbm.at[b, :, pl.ds(s0, K)], in_vmem)

        def t_body(t, carry2):
            par = lax.rem(t, 2)
            tg = ci * SUB + t
            @pl.when(t >= 2)
            def _wait_prev():
                pltpu.make_async_copy(
                    stage2.at[par], out_hbm.at[b, tg - 2], sems.at[par]
                ).wait()

            for a in range(NA):
                ch0 = a * C85 + 4

                @plsc.parallel_loop(0, 81, unroll=8)
                def _uniform(i):
                    ch = ch0 + i
                    v = in_vmem[ch, pl.ds(t * 16, 16)]
                    plsc.store_scatter(stage2.at[par], [base_idx + ch], _sigmoid(v))

            s_vec = s0 + t * 16 + lane
            gxv = (s_vec % G).astype(jnp.float32)
            gyv = (s_vec // G).astype(jnp.float32)
            for a in range(NA):
                o = a * C85
                v0 = in_vmem[o + 0, pl.ds(t * 16, 16)]
                plsc.store_scatter(
                    stage2.at[par], [base_idx + (o + 0)], (_sigmoid(v0) + gxv) * STRIDE
                )
                v1 = in_vmem[o + 1, pl.ds(t * 16, 16)]
                plsc.store_scatter(
                    stage2.at[par], [base_idx + (o + 1)], (_sigmoid(v1) + gyv) * STRIDE
                )
                v2 = in_vmem[o + 2, pl.ds(t * 16, 16)]
                plsc.store_scatter(
                    stage2.at[par], [base_idx + (o + 2)], jnp.exp(v2) * AW[a]
                )
                v3 = in_vmem[o + 3, pl.ds(t * 16, 16)]
                plsc.store_scatter(
                    stage2.at[par], [base_idx + (o + 3)], jnp.exp(v3) * AH[a]
                )

            pltpu.async_copy(stage2.at[par], out_hbm.at[b, tg], sems.at[par])
            return carry2

        lax.fori_loop(0, SUB, t_body, 0)
        pltpu.make_async_copy(
            stage2.at[(SUB - 2) % 2], out_hbm.at[b, ci * SUB + SUB - 2], sems.at[(SUB - 2) % 2]
        ).wait()
        pltpu.make_async_copy(
            stage2.at[(SUB - 1) % 2], out_hbm.at[b, ci * SUB + SUB - 1], sems.at[(SUB - 1) % 2]
        ).wait()
        return carry

    lax.fori_loop(0, NCHUNK, chunk_body, 0)


_sc_run = functools.partial(
    pl.kernel,
    mesh=plsc.VectorSubcoreMesh(core_axis_name="c", subcore_axis_name="s"),
    out_type=jax.ShapeDtypeStruct((B, S // 16, TILE_W), jnp.float32),
    scratch_types=[
        pltpu.VMEM((CH, K), jnp.float32),
        pltpu.VMEM((2, TILE_W), jnp.float32),
        pltpu.SemaphoreType.DMA((2,)),
    ],
    compiler_params=pltpu.CompilerParams(
        use_tc_tiling_on_sc=False, needs_layout_passes=False
    ),
)(_sc_body)


def kernel(x):
    out = _sc_run(x.reshape(B, CH, S))
    return out.reshape(B, S * NA, C85), 0.0


# speedup vs baseline: 6.8273x; 4.1603x over previous
import jax
import jax.numpy as jnp
from jax.experimental import pallas as pl

B = 32
NA = 3
C85 = 85
G = 76
S = G * G
STRIDE = 8.0
AW = (10.0, 16.0, 33.0)
AH = (13.0, 30.0, 23.0)


def _tc_body(x_ref, o_ref):
    v = x_ref[0]
    sig = jax.nn.sigmoid(v)
    ki = jax.lax.broadcasted_iota(jnp.int32, (1, S), 1)
    gx = jnp.mod(ki, G).astype(jnp.float32)
    gy = (ki // G).astype(jnp.float32)
    rows = []
    for a in range(NA):
        o = a * C85
        ex = jnp.exp(v[o + 2 : o + 4])
        rows += [
            (sig[o : o + 1] + gx) * STRIDE,
            (sig[o + 1 : o + 2] + gy) * STRIDE,
            ex[0:1] * AW[a],
            ex[1:2] * AH[a],
            sig[o + 4 : o + C85],
        ]
    act = jnp.concatenate(rows, axis=0)
    o_ref[0] = act.T


def kernel(x):
    x3 = x.reshape(B, NA * C85, S)
    out = pl.pallas_call(
        _tc_body,
        grid=(B,),
        in_specs=[pl.BlockSpec((1, NA * C85, S), lambda b: (b, 0, 0))],
        out_specs=pl.BlockSpec((1, S, NA * C85), lambda b: (b, 0, 0)),
        out_shape=jax.ShapeDtypeStruct((B, S, NA * C85), jnp.float32),
    )(x3)
    return out.reshape(B, S * NA, C85), 0.0
